# Initial kernel scaffold; baseline (speedup 1.0000x reference)
import functools

import jax
import jax.numpy as jnp
from jax import lax
from jax.experimental import pallas as pl
from jax.experimental.pallas import tpu as pltpu
from jax.experimental.pallas import tpu_sc as plsc

B_ROWS = 16384
B_COLS = 26
B_TOTAL = B_ROWS * B_COLS
D = 32
NUM_CORES = 2
NUM_SUBCORES = 16
NW = NUM_CORES * NUM_SUBCORES
B_PER_W = B_TOTAL // NW
CHUNK = 1024
N_CHUNKS = B_PER_W // CHUNK

_mesh = plsc.VectorSubcoreMesh(core_axis_name="c", subcore_axis_name="s")


@functools.partial(
    pl.kernel,
    mesh=_mesh,
    out_type=jax.ShapeDtypeStruct((B_TOTAL, D), jnp.float32),
    scratch_types=[
        pltpu.VMEM((CHUNK,), jnp.int32),
        pltpu.VMEM((CHUNK, D), jnp.float32),
        pltpu.SemaphoreType.DMA,
    ],
)
def _gather_sc(idx_hbm, table_hbm, out_hbm, idx_v, rows_v, sem):
    wid = lax.axis_index("s") * NUM_CORES + lax.axis_index("c")
    base = wid * B_PER_W

    def body(ci, carry):
        off = base + ci * CHUNK
        pltpu.sync_copy(idx_hbm.at[pl.ds(off, CHUNK)], idx_v)
        pltpu.async_copy(table_hbm.at[idx_v], rows_v, sem).wait()
        pltpu.sync_copy(rows_v, out_hbm.at[pl.ds(off, CHUNK)])
        return carry

    lax.fori_loop(0, N_CHUNKS, body, 0)


def kernel(indices, table):
    idx = indices.reshape(-1).astype(jnp.int32)
    out = _gather_sc(idx, table)
    return out.reshape(B_ROWS, B_COLS, D)


# speedup vs baseline: 1.5465x; 1.5465x over previous
import functools

import jax
import jax.numpy as jnp
from jax import lax
from jax.experimental import pallas as pl
from jax.experimental.pallas import tpu as pltpu
from jax.experimental.pallas import tpu_sc as plsc

B_ROWS = 16384
B_COLS = 26
B_TOTAL = B_ROWS * B_COLS
D = 32
NUM_CORES = 2
NUM_SUBCORES = 16
NW = NUM_CORES * NUM_SUBCORES
B_PER_W = B_TOTAL // NW
CHUNK = 1024
N_CHUNKS = B_PER_W // CHUNK

_mesh = plsc.VectorSubcoreMesh(core_axis_name="c", subcore_axis_name="s")


@functools.partial(
    pl.kernel,
    mesh=_mesh,
    out_type=jax.ShapeDtypeStruct((B_TOTAL, D), jnp.float32),
    scratch_types=[
        pltpu.VMEM((CHUNK,), jnp.int32),
        pltpu.VMEM((CHUNK, D), jnp.float32),
        pltpu.SemaphoreType.DMA,
    ],
    compiler_params=pltpu.CompilerParams(use_tc_tiling_on_sc=False),
)
def _gather_sc(idx_hbm, table_hbm, out_hbm, idx_v, rows_v, sem):
    wid = lax.axis_index("s") * NUM_CORES + lax.axis_index("c")
    base = wid * B_PER_W

    def body(ci, carry):
        off = base + ci * CHUNK
        pltpu.sync_copy(idx_hbm.at[pl.ds(off, CHUNK)], idx_v)
        pltpu.async_copy(table_hbm.at[idx_v], rows_v, sem).wait()
        pltpu.sync_copy(rows_v, out_hbm.at[pl.ds(off, CHUNK)])
        return carry

    lax.fori_loop(0, N_CHUNKS, body, 0)


def kernel(indices, table):
    idx = indices.reshape(-1).astype(jnp.int32)
    out = _gather_sc(idx, table)
    return out.reshape(B_ROWS, B_COLS, D)


# speedup vs baseline: 1.5683x; 1.0141x over previous
import functools

import jax
import jax.numpy as jnp
from jax import lax
from jax.experimental import pallas as pl
from jax.experimental.pallas import tpu as pltpu
from jax.experimental.pallas import tpu_sc as plsc

B_ROWS = 16384
B_COLS = 26
B_TOTAL = B_ROWS * B_COLS
D = 32
NUM_CORES = 2
NUM_SUBCORES = 16
NW = NUM_CORES * NUM_SUBCORES
B_PER_W = B_TOTAL // NW
CHUNK = 1664
N_CHUNKS = B_PER_W // CHUNK

_mesh = plsc.VectorSubcoreMesh(core_axis_name="c", subcore_axis_name="s")


@functools.partial(
    pl.kernel,
    mesh=_mesh,
    out_type=jax.ShapeDtypeStruct((B_TOTAL, D), jnp.float32),
    scratch_types=[
        pltpu.VMEM((CHUNK,), jnp.int32),
        pltpu.VMEM((CHUNK,), jnp.int32),
        pltpu.VMEM((CHUNK, D), jnp.float32),
        pltpu.VMEM((CHUNK, D), jnp.float32),
        pltpu.SemaphoreType.DMA,
        pltpu.SemaphoreType.DMA,
        pltpu.SemaphoreType.DMA,
        pltpu.SemaphoreType.DMA,
        pltpu.SemaphoreType.DMA,
        pltpu.SemaphoreType.DMA,
    ],
    compiler_params=pltpu.CompilerParams(use_tc_tiling_on_sc=False),
)
def _gather_sc(idx_hbm, table_hbm, out_hbm,
               idx_v0, idx_v1, rows_v0, rows_v1,
               si0, si1, sg0, sg1, so0, so1):
    wid = lax.axis_index("s") * NUM_CORES + lax.axis_index("c")
    base = wid * B_PER_W
    idx_v = (idx_v0, idx_v1)
    rows_v = (rows_v0, rows_v1)
    sem_i = (si0, si1)
    sem_g = (sg0, sg1)
    sem_o = (so0, so1)

    def off(i):
        return base + i * CHUNK

    cp_i = [None, None]
    cp_g = [None, None]
    cp_o = [None, None]
    cp_i[0] = pltpu.async_copy(idx_hbm.at[pl.ds(off(0), CHUNK)], idx_v[0], sem_i[0])
    if N_CHUNKS > 1:
        cp_i[1] = pltpu.async_copy(idx_hbm.at[pl.ds(off(1), CHUNK)], idx_v[1], sem_i[1])

    for i in range(N_CHUNKS):
        b = i % 2
        if i >= 1:
            p = 1 - b
            cp_g[p].wait()
            cp_o[p] = pltpu.async_copy(
                rows_v[p], out_hbm.at[pl.ds(off(i - 1), CHUNK)], sem_o[p])
            if i + 1 < N_CHUNKS:
                cp_i[p] = pltpu.async_copy(
                    idx_hbm.at[pl.ds(off(i + 1), CHUNK)], idx_v[p], sem_i[p])
        if i >= 2:
            cp_o[b].wait()
        cp_i[b].wait()
        cp_g[b] = pltpu.async_copy(table_hbm.at[idx_v[b]], rows_v[b], sem_g[b])

    last = (N_CHUNKS - 1) % 2
    cp_g[last].wait()
    cp_o[last] = pltpu.async_copy(
        rows_v[last], out_hbm.at[pl.ds(off(N_CHUNKS - 1), CHUNK)], sem_o[last])
    if N_CHUNKS > 1:
        cp_o[1 - last].wait()
    cp_o[last].wait()


def kernel(indices, table):
    idx = indices.reshape(-1).astype(jnp.int32)
    out = _gather_sc(idx, table)
    return out.reshape(B_ROWS, B_COLS, D)


# speedup vs baseline: 2.0005x; 1.2756x over previous
import functools

import jax
import jax.numpy as jnp
from jax import lax
from jax.experimental import pallas as pl
from jax.experimental.pallas import tpu as pltpu
from jax.experimental.pallas import tpu_sc as plsc

B = 16384
F = 26
D = 32
NUM_CORES = 2
NUM_SUBCORES = 16
NW = NUM_CORES * NUM_SUBCORES
BW = B // NW
NBLK = BW // 128

_mesh = plsc.VectorSubcoreMesh(core_axis_name="c", subcore_axis_name="s")


@functools.partial(
    pl.kernel,
    mesh=_mesh,
    out_type=jax.ShapeDtypeStruct((F, D // 8, B // 128, 8, 128), jnp.float32),
    scratch_types=[
        pltpu.VMEM((F, BW), jnp.int32),
        pltpu.VMEM((BW, D), jnp.float32),
        pltpu.VMEM((BW, D), jnp.float32),
        pltpu.VMEM((D, BW), jnp.float32),
        pltpu.VMEM((D, BW), jnp.float32),
        pltpu.SemaphoreType.DMA,
        pltpu.SemaphoreType.DMA,
        pltpu.SemaphoreType.DMA,
        pltpu.SemaphoreType.DMA,
        pltpu.SemaphoreType.DMA,
    ],
    compiler_params=pltpu.CompilerParams(
        use_tc_tiling_on_sc=False, needs_layout_passes=False),
)
def _gather_sc(idx_hbm, table_hbm, y_hbm,
               idx_all, rows0, rows1, bufT0, bufT1,
               sem_i, sg0, sg1, so0, so1):
    wid = lax.axis_index("s") * NUM_CORES + lax.axis_index("c")
    b0 = wid * BW
    blk0 = wid * NBLK
    rows = (rows0, rows1)
    bufT = (bufT0, bufT1)
    sem_g = (sg0, sg1)
    sem_o = (so0, so1)

    pltpu.sync_copy(idx_hbm.at[:, pl.ds(b0, BW)], idx_all)

    lanes = lax.broadcasted_iota(jnp.int32, (16,), 0)
    pat = [(lanes + s) & 15 for s in range(16)]

    def transpose(rows_v, bufT_v):
        def tbody(bb, carry):
            b_vec = lanes + bb * 16
            for h in range(2):
                for s in range(16):
                    d_vec = pat[s] if h == 0 else pat[s] + 16
                    v = plsc.load_gather(rows_v, [b_vec, d_vec])
                    plsc.store_scatter(bufT_v, [d_vec, b_vec], v)
            return carry

        lax.fori_loop(0, BW // 16, tbody, 0)

    def emit_tiles(f, bufT_v, sem):
        def wbody(t, carry):
            db = t // NBLK
            j = t % NBLK
            pltpu.async_copy(
                bufT_v.at[pl.ds(db * 8, 8), pl.ds(j * 128, 128)],
                y_hbm.at[f, db, blk0 + j],
                sem,
            )
            return carry

        lax.fori_loop(0, (D // 8) * NBLK, wbody, 0)

    def sem_wait(buf_v, sem):
        pltpu.make_async_copy(table_hbm.at[pl.ds(0, BW)], buf_v, sem).wait()

    def enqueue_gather(f, p):
        pltpu.async_copy(table_hbm.at[idx_all.at[f]], rows[p], sem_g[p])

    enqueue_gather(0, 0)

    def fbody(t, carry):
        f0 = 2 * t
        enqueue_gather(f0 + 1, 1)
        sem_wait(rows[0], sem_g[0])

        @pl.when(t >= 1)
        def _():
            sem_wait(rows[0], sem_o[0])

        transpose(rows[0], bufT[0])
        emit_tiles(f0, bufT[0], sem_o[0])

        @pl.when(t + 1 < F // 2)
        def _():
            enqueue_gather(f0 + 2, 0)

        sem_wait(rows[1], sem_g[1])

        @pl.when(t >= 1)
        def _():
            sem_wait(rows[1], sem_o[1])

        transpose(rows[1], bufT[1])
        emit_tiles(f0 + 1, bufT[1], sem_o[1])
        return carry

    lax.fori_loop(0, F // 2, fbody, 0)
    sem_wait(rows[0], sem_o[0])
    sem_wait(rows[1], sem_o[1])


def kernel(indices, table):
    idx_t = jnp.transpose(indices).astype(jnp.int32)
    y = _gather_sc(idx_t, table)
    return y.transpose(2, 4, 0, 1, 3).reshape(B, F, D)
